# Initial kernel scaffold; baseline (speedup 1.0000x reference)
import jax
import jax.numpy as jnp
from jax.experimental import pallas as pl


def kernel(edge_index_buys, edge_index_rev_buys, emb_user, emb_item, w1_self_buys, w1_neigh_buys, b1_buys, w1_self_rev, w1_neigh_rev, b1_rev, w2_self_buys, w2_neigh_buys, b2_buys, w2_self_rev, w2_neigh_rev, b2_rev):
    raise NotImplementedError("write your pallas kernel here")


# speedup vs baseline: 3.0422x; 3.0422x over previous
import functools

import jax
import jax.numpy as jnp
from jax import lax
from jax.experimental import pallas as pl
from jax.experimental.pallas import tpu as pltpu
from jax.experimental.pallas import tpu_sc as plsc

_N = 10000
_E = 320000
_D = 128
_TILES = 16
_NSC = 2
_EPT = _E // (_TILES * _NSC)
_CHUNK = 80
_NCHUNK = _EPT // _CHUNK
_ACCPAD = 10240
_RPT = _ACCPAD // _TILES


def _sc_body(with_deg, src_a, dst_a, x_a, src_b, dst_b, x_b, zrows, ones,
             msum_a, msum_b, deg_a, deg_b,
             sidx, didx, rows, ones_v, sem, acc):
  cid = lax.axis_index("c")
  sid = lax.axis_index("s")
  wid = cid * _TILES + sid
  if with_deg:
    pltpu.sync_copy(ones, ones_v)

  def phase(src, dst, x, out, gather):
    pltpu.sync_copy(zrows, acc.at[pl.ds(sid * _RPT, _RPT)])
    plsc.subcore_barrier()

    def chunk(c, carry):
      base = wid * _EPT + c * _CHUNK
      pltpu.sync_copy(dst.at[pl.ds(base, _CHUNK)], didx)
      if gather:
        pltpu.sync_copy(src.at[pl.ds(base, _CHUNK)], sidx)
        pltpu.async_copy(x.at[sidx], rows, sem).wait()
        pltpu.sync_copy(rows, acc.at[didx], add=True)
      else:
        pltpu.sync_copy(ones_v, acc.at[didx], add=True)
      return carry

    lax.fori_loop(0, _NCHUNK, chunk, 0)
    plsc.subcore_barrier()
    pltpu.sync_copy(acc.at[pl.ds(sid * _RPT, _RPT)],
                    out.at[cid, pl.ds(sid * _RPT, _RPT)])

  phase(src_a, dst_a, x_a, msum_a, True)
  phase(src_b, dst_b, x_b, msum_b, True)
  if with_deg:
    phase(src_a, dst_a, x_a, deg_a, False)
    phase(src_b, dst_b, x_b, deg_b, False)


def _make_sc_call(with_deg):
  mesh = plsc.VectorSubcoreMesh(core_axis_name="c", subcore_axis_name="s")
  f32 = jnp.float32
  out_type = [
      jax.ShapeDtypeStruct((_NSC, _ACCPAD, _D), f32),
      jax.ShapeDtypeStruct((_NSC, _ACCPAD, _D), f32),
      jax.ShapeDtypeStruct((_NSC, _ACCPAD, _D), f32),
      jax.ShapeDtypeStruct((_NSC, _ACCPAD, _D), f32),
  ]
  scratch = [
      pltpu.VMEM((_CHUNK,), jnp.int32),
      pltpu.VMEM((_CHUNK,), jnp.int32),
      pltpu.VMEM((_CHUNK, _D), f32),
      pltpu.VMEM((_CHUNK, _D), f32),
      pltpu.SemaphoreType.DMA,
      pltpu.VMEM_SHARED((_ACCPAD, _D), f32),
  ]
  return pl.kernel(functools.partial(_sc_body, with_deg), out_type=out_type,
                   mesh=mesh, scratch_types=scratch)


def _tc_body(relu, x_ref, p0_ref, p1_ref, d0_ref, d1_ref, ws_ref, wn_ref,
             b_ref, o_ref):
  deg = d0_ref[0] + d1_ref[0]
  mean = (p0_ref[0] + p1_ref[0]) / jnp.maximum(deg, 1.0)
  h = jnp.dot(x_ref[...], ws_ref[...], preferred_element_type=jnp.float32)
  h = h + jnp.dot(mean, wn_ref[...], preferred_element_type=jnp.float32)
  h = h + b_ref[...]
  if relu:
    h = jnp.where(h > 0, h, 0.01 * h)
  o_ref[...] = h


_TC_BLK = 1000


def _make_tc_call(relu):
  grid = (_N // _TC_BLK,)
  part = lambda c: pl.BlockSpec((1, _TC_BLK, _D), lambda i, c=c: (c, i, 0))
  return pl.pallas_call(
      functools.partial(_tc_body, relu),
      grid=grid,
      in_specs=[
          pl.BlockSpec((_TC_BLK, _D), lambda i: (i, 0)),
          part(0), part(1), part(0), part(1),
          pl.BlockSpec((_D, _D), lambda i: (0, 0)),
          pl.BlockSpec((_D, _D), lambda i: (0, 0)),
          pl.BlockSpec((1, _D), lambda i: (0, 0)),
      ],
      out_specs=pl.BlockSpec((_TC_BLK, _D), lambda i: (i, 0)),
      out_shape=jax.ShapeDtypeStruct((_N, _D), jnp.float32),
  )


def kernel(edge_index_buys, edge_index_rev_buys, emb_user, emb_item,
           w1_self_buys, w1_neigh_buys, b1_buys,
           w1_self_rev, w1_neigh_rev, b1_rev,
           w2_self_buys, w2_neigh_buys, b2_buys,
           w2_self_rev, w2_neigh_rev, b2_rev):
  f32 = jnp.float32
  src_b, dst_b = edge_index_buys[0], edge_index_buys[1]
  src_r, dst_r = edge_index_rev_buys[0], edge_index_rev_buys[1]
  zrows = jnp.zeros((_RPT, _D), f32)
  ones = jnp.ones((_CHUNK, _D), f32)

  sc1 = _make_sc_call(True)
  msum1_i, msum1_u, degp_i, degp_u = sc1(
      src_b, dst_b, emb_user, src_r, dst_r, emb_item, zrows, ones)

  tc_relu = _make_tc_call(True)
  h1_item = tc_relu(emb_item, msum1_i, msum1_i, degp_i, degp_i,
                    w1_self_buys.T, w1_neigh_buys.T, b1_buys[None, :])
  h1_user = tc_relu(emb_user, msum1_u, msum1_u, degp_u, degp_u,
                    w1_self_rev.T, w1_neigh_rev.T, b1_rev[None, :])

  sc2 = _make_sc_call(False)
  msum2_i, msum2_u, _, _ = sc2(
      src_b, dst_b, h1_user, src_r, dst_r, h1_item, zrows, ones)

  tc_lin = _make_tc_call(False)
  h2_item = tc_lin(h1_item, msum2_i, msum2_i, degp_i, degp_i,
                   w2_self_buys.T, w2_neigh_buys.T, b2_buys[None, :])
  h2_user = tc_lin(h1_user, msum2_u, msum2_u, degp_u, degp_u,
                   w2_self_rev.T, w2_neigh_rev.T, b2_rev[None, :])

  return jnp.concatenate([h2_user, h2_item], axis=0)


# speedup vs baseline: 4.9492x; 1.6268x over previous
import functools

import jax
import jax.numpy as jnp
from jax import lax
from jax.experimental import pallas as pl
from jax.experimental.pallas import tpu as pltpu
from jax.experimental.pallas import tpu_sc as plsc

_N = 10000
_E = 320000
_D = 128
_TILES = 16
_NSC = 2
_EPT = _E // (_TILES * _NSC)
_CHUNK = 80
_NCHUNK = _EPT // _CHUNK
_ACCPAD = 10240
_RPT = _ACCPAD // _TILES


def _sc_body(with_deg, src_a, dst_a, x_a, src_b, dst_b, x_b, zrows, ones,
             msum_a, msum_b, deg_a, deg_b,
             sidx0, didx0, sidx1, didx1, rows0, rows1, ones_v,
             gs0, gs1, isem, acc):
  cid = lax.axis_index("c")
  sid = lax.axis_index("s")
  wid = cid * _TILES + sid
  if with_deg:
    pltpu.sync_copy(ones, ones_v)

  def phase(src, dst, x, out, gather):
    pltpu.sync_copy(zrows, acc.at[pl.ds(sid * _RPT, _RPT)])
    plsc.subcore_barrier()

    def base(c):
      return wid * _EPT + c * _CHUNK

    if gather:
      pltpu.sync_copy(src.at[pl.ds(base(0), _CHUNK)], sidx0)
      pltpu.sync_copy(dst.at[pl.ds(base(0), _CHUNK)], didx0)
      pltpu.async_copy(x.at[sidx0], rows0, gs0)

      def pair(i, carry):
        pltpu.sync_copy(src.at[pl.ds(base(2 * i + 1), _CHUNK)], sidx1)
        pltpu.sync_copy(dst.at[pl.ds(base(2 * i + 1), _CHUNK)], didx1)
        pltpu.make_async_copy(x.at[sidx0], rows0, gs0).wait()
        pltpu.async_copy(x.at[sidx1], rows1, gs1)
        pltpu.sync_copy(rows0, acc.at[didx0], add=True)
        pltpu.sync_copy(src.at[pl.ds(base(2 * i + 2), _CHUNK)], sidx0)
        pltpu.sync_copy(dst.at[pl.ds(base(2 * i + 2), _CHUNK)], didx0)
        pltpu.make_async_copy(x.at[sidx1], rows1, gs1).wait()
        pltpu.async_copy(x.at[sidx0], rows0, gs0)
        pltpu.sync_copy(rows1, acc.at[didx1], add=True)
        return carry

      lax.fori_loop(0, (_NCHUNK - 1) // 2, pair, 0)
      pltpu.make_async_copy(x.at[sidx0], rows0, gs0).wait()
      pltpu.sync_copy(rows0, acc.at[didx0], add=True)
    else:
      pltpu.sync_copy(dst.at[pl.ds(base(0), _CHUNK)], didx0)

      def dpair(i, carry):
        a = pltpu.async_copy(dst.at[pl.ds(base(2 * i + 1), _CHUNK)], didx1,
                             isem)
        pltpu.sync_copy(ones_v, acc.at[didx0], add=True)
        a.wait()
        b = pltpu.async_copy(dst.at[pl.ds(base(2 * i + 2), _CHUNK)], didx0,
                             isem)
        pltpu.sync_copy(ones_v, acc.at[didx1], add=True)
        b.wait()
        return carry

      lax.fori_loop(0, (_NCHUNK - 1) // 2, dpair, 0)
      pltpu.sync_copy(ones_v, acc.at[didx0], add=True)

    plsc.subcore_barrier()
    pltpu.sync_copy(acc.at[pl.ds(sid * _RPT, _RPT)],
                    out.at[cid, pl.ds(sid * _RPT, _RPT)])

  phase(src_a, dst_a, x_a, msum_a, True)
  phase(src_b, dst_b, x_b, msum_b, True)
  if with_deg:
    phase(src_a, dst_a, x_a, deg_a, False)
    phase(src_b, dst_b, x_b, deg_b, False)


def _make_sc_call(with_deg):
  mesh = plsc.VectorSubcoreMesh(core_axis_name="c", subcore_axis_name="s")
  f32 = jnp.float32
  out_type = [
      jax.ShapeDtypeStruct((_NSC, _ACCPAD, _D), f32),
      jax.ShapeDtypeStruct((_NSC, _ACCPAD, _D), f32),
      jax.ShapeDtypeStruct((_NSC, _ACCPAD, _D), f32),
      jax.ShapeDtypeStruct((_NSC, _ACCPAD, _D), f32),
  ]
  scratch = [
      pltpu.VMEM((_CHUNK,), jnp.int32),
      pltpu.VMEM((_CHUNK,), jnp.int32),
      pltpu.VMEM((_CHUNK,), jnp.int32),
      pltpu.VMEM((_CHUNK,), jnp.int32),
      pltpu.VMEM((_CHUNK, _D), f32),
      pltpu.VMEM((_CHUNK, _D), f32),
      pltpu.VMEM((_CHUNK, _D), f32),
      pltpu.SemaphoreType.DMA,
      pltpu.SemaphoreType.DMA,
      pltpu.SemaphoreType.DMA,
      pltpu.VMEM_SHARED((_ACCPAD, _D), f32),
  ]
  return pl.kernel(functools.partial(_sc_body, with_deg), out_type=out_type,
                   mesh=mesh, scratch_types=scratch)


def _tc_body(relu, x_ref, p0_ref, p1_ref, d0_ref, d1_ref, ws_ref, wn_ref,
             b_ref, o_ref):
  deg = d0_ref[0] + d1_ref[0]
  mean = (p0_ref[0] + p1_ref[0]) / jnp.maximum(deg, 1.0)
  h = jnp.dot(x_ref[...], ws_ref[...], preferred_element_type=jnp.float32)
  h = h + jnp.dot(mean, wn_ref[...], preferred_element_type=jnp.float32)
  h = h + b_ref[...]
  if relu:
    h = jnp.where(h > 0, h, 0.01 * h)
  o_ref[...] = h


_TC_BLK = 1000


def _make_tc_call(relu):
  grid = (_N // _TC_BLK,)
  part = lambda c: pl.BlockSpec((1, _TC_BLK, _D), lambda i, c=c: (c, i, 0))
  return pl.pallas_call(
      functools.partial(_tc_body, relu),
      grid=grid,
      in_specs=[
          pl.BlockSpec((_TC_BLK, _D), lambda i: (i, 0)),
          part(0), part(1), part(0), part(1),
          pl.BlockSpec((_D, _D), lambda i: (0, 0)),
          pl.BlockSpec((_D, _D), lambda i: (0, 0)),
          pl.BlockSpec((1, _D), lambda i: (0, 0)),
      ],
      out_specs=pl.BlockSpec((_TC_BLK, _D), lambda i: (i, 0)),
      out_shape=jax.ShapeDtypeStruct((_N, _D), jnp.float32),
  )


def kernel(edge_index_buys, edge_index_rev_buys, emb_user, emb_item,
           w1_self_buys, w1_neigh_buys, b1_buys,
           w1_self_rev, w1_neigh_rev, b1_rev,
           w2_self_buys, w2_neigh_buys, b2_buys,
           w2_self_rev, w2_neigh_rev, b2_rev):
  f32 = jnp.float32
  src_b, dst_b = edge_index_buys[0], edge_index_buys[1]
  src_r, dst_r = edge_index_rev_buys[0], edge_index_rev_buys[1]
  zrows = jnp.zeros((_RPT, _D), f32)
  ones = jnp.ones((_CHUNK, _D), f32)

  sc1 = _make_sc_call(True)
  msum1_i, msum1_u, degp_i, degp_u = sc1(
      src_b, dst_b, emb_user, src_r, dst_r, emb_item, zrows, ones)

  tc_relu = _make_tc_call(True)
  h1_item = tc_relu(emb_item, msum1_i, msum1_i, degp_i, degp_i,
                    w1_self_buys.T, w1_neigh_buys.T, b1_buys[None, :])
  h1_user = tc_relu(emb_user, msum1_u, msum1_u, degp_u, degp_u,
                    w1_self_rev.T, w1_neigh_rev.T, b1_rev[None, :])

  sc2 = _make_sc_call(False)
  msum2_i, msum2_u, _, _ = sc2(
      src_b, dst_b, h1_user, src_r, dst_r, h1_item, zrows, ones)

  tc_lin = _make_tc_call(False)
  h2_item = tc_lin(h1_item, msum2_i, msum2_i, degp_i, degp_i,
                   w2_self_buys.T, w2_neigh_buys.T, b2_buys[None, :])
  h2_user = tc_lin(h1_user, msum2_u, msum2_u, degp_u, degp_u,
                   w2_self_rev.T, w2_neigh_rev.T, b2_rev[None, :])

  return jnp.concatenate([h2_user, h2_item], axis=0)
